# baseline (speedup 1.0000x reference)
import functools

import jax
import jax.numpy as jnp
from jax import lax
from jax.experimental import pallas as pl
from jax.experimental.pallas import tpu as pltpu
from jax.experimental.pallas import tpu_sc as plsc


DIM = 1024
LANES = 16

_NUM_WORKERS = 32
_CHUNK_ROWS = 16
_CHUNK_WORDS = _CHUNK_ROWS * DIM
_UNROLL = 8


def _sc_body(x_hbm, emb_hbm, out_hbm, ebuf, xbuf, batch, rows_per_worker):
    wid = lax.axis_index("s") * 2 + lax.axis_index("c")
    word0 = wid * (rows_per_worker * DIM)
    num_chunks = rows_per_worker // _CHUNK_ROWS
    groups = _CHUNK_WORDS // (LANES * _UNROLL)

    def chunk_loop(c, _):
        off = word0 + c * _CHUNK_WORDS
        pltpu.sync_copy(emb_hbm.at[pl.ds(off, _CHUNK_WORDS)], ebuf)

        def batch_loop(b, _):
            pltpu.sync_copy(x_hbm.at[b, pl.ds(off, _CHUNK_WORDS)], xbuf)

            def vec_loop(g, _):
                base = g * (LANES * _UNROLL)
                for u in range(_UNROLL):
                    sl = pl.ds(base + u * LANES, LANES)
                    xbuf[sl] = xbuf[sl] + ebuf[sl]
                return 0

            lax.fori_loop(0, groups, vec_loop, 0, unroll=False)
            pltpu.sync_copy(xbuf, out_hbm.at[b, pl.ds(off, _CHUNK_WORDS)])
            return 0

        lax.fori_loop(0, batch, batch_loop, 0, unroll=False)
        return 0

    lax.fori_loop(0, num_chunks, chunk_loop, 0, unroll=False)


def _sc_add(x2, embf):
    batch, n = x2.shape
    rows_per_worker = (n // DIM) // _NUM_WORKERS
    mesh = plsc.VectorSubcoreMesh(core_axis_name="c", subcore_axis_name="s")
    body = functools.partial(
        _sc_body, batch=batch, rows_per_worker=rows_per_worker
    )
    return pl.kernel(
        body,
        out_type=jax.ShapeDtypeStruct((batch, n), jnp.float32),
        mesh=mesh,
        scratch_types=[
            pltpu.VMEM((_CHUNK_WORDS,), jnp.float32),
            pltpu.VMEM((_CHUNK_WORDS,), jnp.float32),
        ],
    )(x2, embf)


def kernel(x, emb):
    batch, seq, dim = x.shape
    x2 = x.reshape(batch, seq * dim)
    embf = emb[:seq].reshape(seq * dim)
    out2 = _sc_add(x2, embf)
    return out2.reshape(batch, seq, dim)


# speedup vs baseline: 1.2994x; 1.2994x over previous
import functools

import jax
import jax.numpy as jnp
from jax import lax
from jax.experimental import pallas as pl
from jax.experimental.pallas import tpu as pltpu
from jax.experimental.pallas import tpu_sc as plsc


DIM = 1024
LANES = 16

_NUM_WORKERS = 32
_CHUNK_ROWS = 16
_CW = _CHUNK_ROWS * DIM
_UNROLL = 8


def _sc_body(x_hbm, emb_hbm, out_hbm,
             ebuf0, ebuf1, xbuf0, xbuf1,
             esem0, esem1, lsem0, lsem1, ssem0, ssem1,
             batch, rows_per_worker):
    wid = lax.axis_index("s") * 2 + lax.axis_index("c")
    word0 = wid * (rows_per_worker * DIM)
    nc = rows_per_worker // _CHUNK_ROWS
    groups = _CW // (LANES * _UNROLL)
    ebuf = (ebuf0, ebuf1)
    xbuf = (xbuf0, xbuf1)
    esem = (esem0, esem1)
    lsem = (lsem0, lsem1)
    ssem = (ssem0, ssem1)

    def e_src(c):
        return emb_hbm.at[pl.ds(word0 + c * _CW, _CW)]

    def x_src(c, b):
        return x_hbm.at[b, pl.ds(word0 + c * _CW, _CW)]

    def o_dst(c, b):
        return out_hbm.at[b, pl.ds(word0 + c * _CW, _CW)]

    pltpu.make_async_copy(e_src(0), ebuf[0], esem[0]).start()
    pltpu.make_async_copy(x_src(0, 0), xbuf[0], lsem[0]).start()

    def chunk_pair(c2, _):
        for cc in range(2):
            c = c2 * 2 + cc
            pltpu.make_async_copy(e_src(c), ebuf[cc], esem[cc]).wait()
            if cc == 0:
                pltpu.make_async_copy(e_src(c + 1), ebuf[1], esem[1]).start()
            else:
                @pl.when(c2 < nc // 2 - 1)
                def _():
                    pltpu.make_async_copy(e_src(c + 1), ebuf[0], esem[0]).start()

            for b in range(batch):
                p = b % 2
                q = 1 - p
                pltpu.make_async_copy(x_src(c, b), xbuf[p], lsem[p]).wait()
                nb = (b + 1) % batch
                ncc = c + 1 if b == batch - 1 else c
                pb = (b - 1) % batch
                pc = c if b > 0 else c - 1
                if cc == 0 and b == 0:
                    @pl.when(c2 > 0)
                    def _():
                        pltpu.make_async_copy(xbuf[q], o_dst(pc, pb), ssem[q]).wait()

                    pltpu.make_async_copy(x_src(ncc, nb), xbuf[q], lsem[q]).start()
                else:
                    pltpu.make_async_copy(xbuf[q], o_dst(pc, pb), ssem[q]).wait()

                    @pl.when(ncc < nc)
                    def _():
                        pltpu.make_async_copy(x_src(ncc, nb), xbuf[q], lsem[q]).start()

                def vec_loop(g, _, p=p, cc=cc):
                    base = g * (LANES * _UNROLL)
                    for u in range(_UNROLL):
                        sl = pl.ds(base + u * LANES, LANES)
                        xbuf[p][sl] = xbuf[p][sl] + ebuf[cc][sl]
                    return 0

                lax.fori_loop(0, groups, vec_loop, 0, unroll=False)
                pltpu.make_async_copy(xbuf[p], o_dst(c, b), ssem[p]).start()
        return 0

    lax.fori_loop(0, nc // 2, chunk_pair, 0, unroll=False)

    pltpu.make_async_copy(xbuf[1], o_dst(nc - 1, batch - 1), ssem[1]).wait()


def _sc_add(x2, embf):
    batch, n = x2.shape
    rows_per_worker = (n // DIM) // _NUM_WORKERS
    mesh = plsc.VectorSubcoreMesh(core_axis_name="c", subcore_axis_name="s")
    body = functools.partial(
        _sc_body, batch=batch, rows_per_worker=rows_per_worker
    )
    return pl.kernel(
        body,
        out_type=jax.ShapeDtypeStruct((batch, n), jnp.float32),
        mesh=mesh,
        scratch_types=[
            pltpu.VMEM((_CW,), jnp.float32),
            pltpu.VMEM((_CW,), jnp.float32),
            pltpu.VMEM((_CW,), jnp.float32),
            pltpu.VMEM((_CW,), jnp.float32),
            pltpu.SemaphoreType.DMA,
            pltpu.SemaphoreType.DMA,
            pltpu.SemaphoreType.DMA,
            pltpu.SemaphoreType.DMA,
            pltpu.SemaphoreType.DMA,
            pltpu.SemaphoreType.DMA,
        ],
    )(x2, embf)


def kernel(x, emb):
    batch, seq, dim = x.shape
    x2 = x.reshape(batch, seq * dim)
    embf = emb[:seq].reshape(seq * dim)
    out2 = _sc_add(x2, embf)
    return out2.reshape(batch, seq, dim)


# speedup vs baseline: 3.2201x; 2.4782x over previous
import functools

import jax
import jax.numpy as jnp
from jax import lax
from jax.experimental import pallas as pl
from jax.experimental.pallas import tpu as pltpu
from jax.experimental.pallas import tpu_sc as plsc


DIM = 1024
LANES = 16

_NUM_WORKERS = 32
_CHUNK_ROWS = 16


def _sc_body(x_hbm, emb_hbm, out_hbm,
             ebuf0, ebuf1, xbuf0, xbuf1,
             esem0, esem1, lsem0, lsem1, ssem0, ssem1,
             batch, rows_per_worker):
    wid = lax.axis_index("s") * 2 + lax.axis_index("c")
    row0 = wid * rows_per_worker
    nc = rows_per_worker // _CHUNK_ROWS
    ebuf = (ebuf0, ebuf1)
    xbuf = (xbuf0, xbuf1)
    esem = (esem0, esem1)
    lsem = (lsem0, lsem1)
    ssem = (ssem0, ssem1)

    def e_src(c):
        return emb_hbm.at[pl.ds(row0 + c * _CHUNK_ROWS, _CHUNK_ROWS), :]

    def x_src(c, b):
        return x_hbm.at[b, pl.ds(row0 + c * _CHUNK_ROWS, _CHUNK_ROWS), :]

    def o_dst(c, b):
        return out_hbm.at[b, pl.ds(row0 + c * _CHUNK_ROWS, _CHUNK_ROWS), :]

    pltpu.make_async_copy(e_src(0), ebuf[0], esem[0]).start()
    pltpu.make_async_copy(x_src(0, 0), xbuf[0], lsem[0]).start()

    def chunk_pair(c2, _):
        for cc in range(2):
            c = c2 * 2 + cc
            pltpu.make_async_copy(e_src(c), ebuf[cc], esem[cc]).wait()
            if cc == 0:
                pltpu.make_async_copy(e_src(c + 1), ebuf[1], esem[1]).start()
            else:
                @pl.when(c2 < nc // 2 - 1)
                def _():
                    pltpu.make_async_copy(e_src(c + 1), ebuf[0], esem[0]).start()

            for b in range(batch):
                p = b % 2
                q = 1 - p
                pltpu.make_async_copy(x_src(c, b), xbuf[p], lsem[p]).wait()
                nb = (b + 1) % batch
                ncc = c + 1 if b == batch - 1 else c
                pb = (b - 1) % batch
                pc = c if b > 0 else c - 1
                if cc == 0 and b == 0:
                    @pl.when(c2 > 0)
                    def _():
                        pltpu.make_async_copy(xbuf[q], o_dst(pc, pb), ssem[q]).wait()

                    pltpu.make_async_copy(x_src(ncc, nb), xbuf[q], lsem[q]).start()
                else:
                    pltpu.make_async_copy(xbuf[q], o_dst(pc, pb), ssem[q]).wait()

                    @pl.when(ncc < nc)
                    def _():
                        pltpu.make_async_copy(x_src(ncc, nb), xbuf[q], lsem[q]).start()

                def vec_loop(r, _, p=p, cc=cc):
                    for u in range(DIM // LANES):
                        sl = pl.ds(u * LANES, LANES)
                        xbuf[p][r, sl] = xbuf[p][r, sl] + ebuf[cc][r, sl]
                    return 0

                lax.fori_loop(0, _CHUNK_ROWS, vec_loop, 0, unroll=False)
                pltpu.make_async_copy(xbuf[p], o_dst(c, b), ssem[p]).start()
        return 0

    lax.fori_loop(0, nc // 2, chunk_pair, 0, unroll=False)

    pltpu.make_async_copy(xbuf[1], o_dst(nc - 1, batch - 1), ssem[1]).wait()


def kernel(x, emb):
    batch, seq, dim = x.shape
    rows_per_worker = seq // _NUM_WORKERS
    mesh = plsc.VectorSubcoreMesh(core_axis_name="c", subcore_axis_name="s")
    body = functools.partial(
        _sc_body, batch=batch, rows_per_worker=rows_per_worker
    )
    return pl.kernel(
        body,
        out_type=jax.ShapeDtypeStruct((batch, seq, dim), jnp.float32),
        mesh=mesh,
        scratch_types=[
            pltpu.VMEM((_CHUNK_ROWS, DIM), jnp.float32),
            pltpu.VMEM((_CHUNK_ROWS, DIM), jnp.float32),
            pltpu.VMEM((_CHUNK_ROWS, DIM), jnp.float32),
            pltpu.VMEM((_CHUNK_ROWS, DIM), jnp.float32),
            pltpu.SemaphoreType.DMA,
            pltpu.SemaphoreType.DMA,
            pltpu.SemaphoreType.DMA,
            pltpu.SemaphoreType.DMA,
            pltpu.SemaphoreType.DMA,
            pltpu.SemaphoreType.DMA,
        ],
    )(x, emb[:seq])


# speedup vs baseline: 3.8564x; 1.1976x over previous
import functools

import jax
import jax.numpy as jnp
from jax import lax
from jax.experimental import pallas as pl
from jax.experimental.pallas import tpu as pltpu
from jax.experimental.pallas import tpu_sc as plsc


DIM = 1024
LANES = 16

_NUM_WORKERS = 32
_CHUNK_ROWS = 16


def _sc_body(x_hbm, emb_hbm, out_hbm,
             ebuf0, ebuf1, xbuf0, xbuf1, obuf0, obuf1,
             esem0, esem1, lsem0, lsem1, ssem0, ssem1,
             batch, rows_per_worker):
    wid = lax.axis_index("s") * 2 + lax.axis_index("c")
    row0 = wid * rows_per_worker
    nc = rows_per_worker // _CHUNK_ROWS
    ebuf = (ebuf0, ebuf1)
    xbuf = (xbuf0, xbuf1)
    obuf = (obuf0, obuf1)
    esem = (esem0, esem1)
    lsem = (lsem0, lsem1)
    ssem = (ssem0, ssem1)

    def e_src(c):
        return emb_hbm.at[pl.ds(row0 + c * _CHUNK_ROWS, _CHUNK_ROWS), :]

    def x_src(c, b):
        return x_hbm.at[b, pl.ds(row0 + c * _CHUNK_ROWS, _CHUNK_ROWS), :]

    def o_dst(c, b):
        return out_hbm.at[b, pl.ds(row0 + c * _CHUNK_ROWS, _CHUNK_ROWS), :]

    pltpu.make_async_copy(e_src(0), ebuf[0], esem[0]).start()
    pltpu.make_async_copy(x_src(0, 0), xbuf[0], lsem[0]).start()
    pltpu.make_async_copy(x_src(0, 1), xbuf[1], lsem[1]).start()

    def chunk_pair(c2, _):
        for cc in range(2):
            c = c2 * 2 + cc
            pltpu.make_async_copy(e_src(c), ebuf[cc], esem[cc]).wait()
            if cc == 0:
                pltpu.make_async_copy(e_src(c + 1), ebuf[1], esem[1]).start()
            else:
                @pl.when(c2 < nc // 2 - 1)
                def _():
                    pltpu.make_async_copy(e_src(c + 1), ebuf[0], esem[0]).start()

            for b in range(batch):
                p = b % 2
                pltpu.make_async_copy(x_src(c, b), xbuf[p], lsem[p]).wait()

                pb2 = (b - 2) % batch
                pc2 = c if b >= 2 else c - 1
                if cc == 0 and b < 2:
                    @pl.when(c2 > 0)
                    def _():
                        pltpu.make_async_copy(obuf[p], o_dst(pc2, pb2), ssem[p]).wait()
                else:
                    pltpu.make_async_copy(obuf[p], o_dst(pc2, pb2), ssem[p]).wait()

                @functools.partial(plsc.parallel_loop, 0, _CHUNK_ROWS)
                def _(r, p=p, cc=cc):
                    for u in range(DIM // LANES):
                        sl = pl.ds(u * LANES, LANES)
                        obuf[p][r, sl] = xbuf[p][r, sl] + ebuf[cc][r, sl]

                pltpu.make_async_copy(obuf[p], o_dst(c, b), ssem[p]).start()
                nb = (b + 2) % batch
                ncc = c + 1 if b >= batch - 2 else c
                @pl.when(ncc < nc)
                def _():
                    pltpu.make_async_copy(x_src(ncc, nb), xbuf[p], lsem[p]).start()
        return 0

    lax.fori_loop(0, nc // 2, chunk_pair, 0, unroll=False)

    pltpu.make_async_copy(obuf[0], o_dst(nc - 1, batch - 2), ssem[0]).wait()
    pltpu.make_async_copy(obuf[1], o_dst(nc - 1, batch - 1), ssem[1]).wait()


def kernel(x, emb):
    batch, seq, dim = x.shape
    rows_per_worker = seq // _NUM_WORKERS
    mesh = plsc.VectorSubcoreMesh(core_axis_name="c", subcore_axis_name="s")
    body = functools.partial(
        _sc_body, batch=batch, rows_per_worker=rows_per_worker
    )
    return pl.kernel(
        body,
        out_type=jax.ShapeDtypeStruct((batch, seq, dim), jnp.float32),
        mesh=mesh,
        scratch_types=[
            pltpu.VMEM((_CHUNK_ROWS, DIM), jnp.float32),
            pltpu.VMEM((_CHUNK_ROWS, DIM), jnp.float32),
            pltpu.VMEM((_CHUNK_ROWS, DIM), jnp.float32),
            pltpu.VMEM((_CHUNK_ROWS, DIM), jnp.float32),
            pltpu.VMEM((_CHUNK_ROWS, DIM), jnp.float32),
            pltpu.VMEM((_CHUNK_ROWS, DIM), jnp.float32),
            pltpu.SemaphoreType.DMA,
            pltpu.SemaphoreType.DMA,
            pltpu.SemaphoreType.DMA,
            pltpu.SemaphoreType.DMA,
            pltpu.SemaphoreType.DMA,
            pltpu.SemaphoreType.DMA,
        ],
    )(x, emb[:seq])
